# baseline (speedup 1.0000x reference)
import functools

import jax
import jax.numpy as jnp
from jax import lax
from jax.experimental import pallas as pl
from jax.experimental.pallas import tpu as pltpu
from jax.experimental.pallas import tpu_sc as plsc

_NC, _NS = 2, 16
_NW = _NC * _NS
_S = 5
_BLK = 640


def _tc_transpose(x2d):
    C, E = x2d.shape
    eye = jnp.eye(C, dtype=jnp.bfloat16)

    def body(x_ref, i_ref, o_ref):
        o_ref[...] = lax.dot_general(
            x_ref[...].astype(jnp.bfloat16), i_ref[...],
            (((0,), (0,)), ((), ())),
            preferred_element_type=jnp.float32,
        )

    return pl.pallas_call(
        body,
        grid=(E // _BLK,),
        in_specs=[
            pl.BlockSpec((C, _BLK), lambda i: (0, i)),
            pl.BlockSpec((C, C), lambda i: (0, 0)),
        ],
        out_specs=pl.BlockSpec((_BLK, C), lambda i: (i, 0)),
        out_shape=jax.ShapeDtypeStruct((E, C), jnp.float32),
    )(x2d, eye)


def _sc_gather(xT, i1, i2, i3, i4, e0, Es):
    E, C = xT.shape
    CH = 128
    n_chunks = Es // CH
    n_lo = n_chunks // _NW
    extra = n_chunks - n_lo * _NW
    max_ch = n_lo + (1 if extra else 0)

    mesh = plsc.VectorSubcoreMesh(core_axis_name="c", subcore_axis_name="s")

    @functools.partial(
        pl.kernel,
        mesh=mesh,
        out_type=jax.ShapeDtypeStruct((4, Es, C), jnp.float32),
        scratch_types=[
            pltpu.VMEM((max_ch * CH,), jnp.int32),
            pltpu.VMEM((max_ch * CH,), jnp.int32),
            pltpu.VMEM((max_ch * CH,), jnp.int32),
            pltpu.VMEM((max_ch * CH,), jnp.int32),
            pltpu.VMEM((4, CH, C), jnp.float32),
            pltpu.SemaphoreType.DMA,
        ],
    )
    def k(xT_hbm, i1_hbm, i2_hbm, i3_hbm, i4_hbm, out_hbm,
          iv1, iv2, iv3, iv4, rows_v, sem):
        wid = lax.axis_index("s") * _NC + lax.axis_index("c")
        start_ch = n_lo * wid + jnp.minimum(wid, extra)
        n_ch = n_lo + jnp.where(wid < extra, 1, 0)
        base = pl.multiple_of(start_ch * CH, CH)
        idx_vs = (iv1, iv2, iv3, iv4)
        for j4, ik in enumerate((i1_hbm, i2_hbm, i3_hbm, i4_hbm)):
            pltpu.sync_copy(ik.at[pl.ds(e0 + base, n_lo * CH)],
                            idx_vs[j4].at[pl.ds(0, n_lo * CH)])
        if extra:
            @pl.when(wid < extra)
            def _():
                for j4, ik in enumerate((i1_hbm, i2_hbm, i3_hbm, i4_hbm)):
                    pltpu.sync_copy(
                        ik.at[pl.ds(e0 + base + n_lo * CH, CH)],
                        idx_vs[j4].at[pl.ds(n_lo * CH, CH)])

        def chunk(j, carry):
            off = pl.multiple_of(j * CH, CH)
            cps = [
                pltpu.async_copy(
                    xT_hbm.at[idx_vs[j4].at[pl.ds(off, CH)]], rows_v.at[j4],
                    sem,
                )
                for j4 in range(4)
            ]
            for cp in cps:
                cp.wait()
            for j4 in range(4):
                pltpu.sync_copy(
                    rows_v.at[j4], out_hbm.at[j4, pl.ds(base + off, CH)]
                )
            return carry

        lax.fori_loop(0, n_ch, chunk, 0)

    return k(xT, i1, i2, i3, i4)


def _tc_conv(xT, f4, w0, w14, b2, e0, Es):
    E, C = xT.shape
    O = w0.shape[0]
    blk0 = e0 // _BLK

    def body(x_ref, f_ref, w0_ref, w14_ref, b_ref, o_ref):
        bf = jnp.bfloat16
        xb = x_ref[...].astype(bf)
        f1 = f_ref[0]
        f2 = f_ref[1]
        f3 = f_ref[2]
        f4_ = f_ref[3]
        s13 = (f1 + f3).astype(bf)
        s24 = (f2 + f4_).astype(bf)
        d13 = jnp.abs(f1 - f3).astype(bf)
        d24 = jnp.abs(f2 - f4_).astype(bf)
        w14 = w14_ref[...]

        def mm(wk, feat):
            return lax.dot_general(
                wk, feat, (((1,), (1,)), ((), ())),
                preferred_element_type=jnp.float32,
            )

        acc = (mm(w0_ref[...], xb) + mm(w14[0], s13) + mm(w14[1], s24)
               + mm(w14[2], d13) + mm(w14[3], d24))
        o_ref[...] = acc + b_ref[...]

    return pl.pallas_call(
        body,
        grid=(Es // _BLK,),
        in_specs=[
            pl.BlockSpec((_BLK, C), lambda i: (i + blk0, 0)),
            pl.BlockSpec((4, _BLK, C), lambda i: (0, i, 0)),
            pl.BlockSpec((O, C), lambda i: (0, 0)),
            pl.BlockSpec((4, O, C), lambda i: (0, 0, 0)),
            pl.BlockSpec((O, 1), lambda i: (0, 0)),
        ],
        out_specs=pl.BlockSpec((O, _BLK), lambda i: (0, i)),
        out_shape=jax.ShapeDtypeStruct((O, Es), jnp.float32),
    )(xT, f4, w0, w14, b2)


def kernel(x, gemm_edges, W, b):
    x2d = x[0, :, :, 0]
    E = x2d.shape[1]
    xT = _tc_transpose(x2d)
    ge = gemm_edges[0].astype(jnp.int32)
    i1, i2, i3, i4 = ge[:, 0], ge[:, 1], ge[:, 2], ge[:, 3]
    w0 = W[:, :, 0, 0].astype(jnp.bfloat16)
    w14 = jnp.transpose(
        W[:, :, 0, 1:], (2, 0, 1)).astype(jnp.bfloat16)
    b2 = b[:, None]
    Es = E // _S
    f4s = [_sc_gather(xT, i1, i2, i3, i4, s * Es, Es) for s in range(_S)]
    outs = [_tc_conv(xT, f4s[s], w0, w14, b2, s * Es, Es)
            for s in range(_S)]
    out = jnp.concatenate(outs, axis=1) if _S > 1 else outs[0]
    return out[None, :, :, None]


# speedup vs baseline: 1.3104x; 1.3104x over previous
import functools

import jax
import jax.numpy as jnp
from jax import lax
from jax.experimental import pallas as pl
from jax.experimental.pallas import tpu as pltpu
from jax.experimental.pallas import tpu_sc as plsc

_NC, _NS = 2, 16
_NW = _NC * _NS
_S = 5
_BLK = 640


def _tc_transpose(x2d):
    C, E = x2d.shape
    eye = jnp.eye(C, dtype=jnp.bfloat16)

    def body(x_ref, i_ref, o_ref):
        o_ref[...] = lax.dot_general(
            x_ref[...].astype(jnp.bfloat16), i_ref[...],
            (((0,), (0,)), ((), ())),
            preferred_element_type=jnp.float32,
        )

    return pl.pallas_call(
        body,
        grid=(E // _BLK,),
        in_specs=[
            pl.BlockSpec((C, _BLK), lambda i: (0, i)),
            pl.BlockSpec((C, C), lambda i: (0, 0)),
        ],
        out_specs=pl.BlockSpec((_BLK, C), lambda i: (i, 0)),
        out_shape=jax.ShapeDtypeStruct((E, C), jnp.float32),
    )(x2d, eye)


def _sc_gather(xT, i1, i2, i3, i4, e0, Es):
    E, C = xT.shape
    CH = 128
    n_chunks = Es // CH
    n_lo = n_chunks // _NW
    extra = n_chunks - n_lo * _NW
    max_ch = n_lo + (1 if extra else 0)

    mesh = plsc.VectorSubcoreMesh(core_axis_name="c", subcore_axis_name="s")

    @functools.partial(
        pl.kernel,
        mesh=mesh,
        out_type=jax.ShapeDtypeStruct((4, Es, C), jnp.float32),
        scratch_types=[
            pltpu.VMEM((max_ch * CH,), jnp.int32),
            pltpu.VMEM((max_ch * CH,), jnp.int32),
            pltpu.VMEM((max_ch * CH,), jnp.int32),
            pltpu.VMEM((max_ch * CH,), jnp.int32),
            pltpu.VMEM((4, CH, C), jnp.float32),
            pltpu.SemaphoreType.DMA,
        ],
    )
    def k(xT_hbm, i1_hbm, i2_hbm, i3_hbm, i4_hbm, out_hbm,
          iv1, iv2, iv3, iv4, rows_v, sem):
        wid = lax.axis_index("s") * _NC + lax.axis_index("c")
        start_ch = n_lo * wid + jnp.minimum(wid, extra)
        n_ch = n_lo + jnp.where(wid < extra, 1, 0)
        base = pl.multiple_of(start_ch * CH, CH)
        idx_vs = (iv1, iv2, iv3, iv4)
        for j4, ik in enumerate((i1_hbm, i2_hbm, i3_hbm, i4_hbm)):
            pltpu.sync_copy(ik.at[pl.ds(e0 + base, n_lo * CH)],
                            idx_vs[j4].at[pl.ds(0, n_lo * CH)])
        if extra:
            @pl.when(wid < extra)
            def _():
                for j4, ik in enumerate((i1_hbm, i2_hbm, i3_hbm, i4_hbm)):
                    pltpu.sync_copy(
                        ik.at[pl.ds(e0 + base + n_lo * CH, CH)],
                        idx_vs[j4].at[pl.ds(n_lo * CH, CH)])

        def chunk(j, carry):
            off = pl.multiple_of(j * CH, CH)
            cps = [
                pltpu.async_copy(
                    xT_hbm.at[idx_vs[j4].at[pl.ds(off, CH)]], rows_v.at[j4],
                    sem,
                )
                for j4 in range(4)
            ]
            for cp in cps:
                cp.wait()
            for j4 in range(4):
                pltpu.sync_copy(
                    rows_v.at[j4], out_hbm.at[j4, pl.ds(base + off, CH)]
                )
            return carry

        lax.fori_loop(0, n_ch, chunk, 0)

    return k(xT, i1, i2, i3, i4)


def _tc_conv(xT, f4, w0, w14, b2, e0, Es):
    E, C = xT.shape
    O = w0.shape[0]
    blk0 = e0 // _BLK

    def body(x_ref, f_ref, w0_ref, w14_ref, b_ref, o_ref):
        bf = jnp.bfloat16
        xb = x_ref[...].astype(bf)
        f1 = f_ref[0]
        f2 = f_ref[1]
        f3 = f_ref[2]
        f4_ = f_ref[3]
        s13 = (f1 + f3).astype(bf)
        s24 = (f2 + f4_).astype(bf)
        d13 = jnp.abs(f1 - f3).astype(bf)
        d24 = jnp.abs(f2 - f4_).astype(bf)
        w14 = w14_ref[...]

        def mm(wk, feat):
            return lax.dot_general(
                wk, feat, (((1,), (1,)), ((), ())),
                preferred_element_type=jnp.float32,
            )

        acc = (mm(w0_ref[...], xb) + mm(w14[0], s13) + mm(w14[1], s24)
               + mm(w14[2], d13) + mm(w14[3], d24))
        o_ref[...] = acc + b_ref[...]

    return pl.pallas_call(
        body,
        grid=(Es // _BLK,),
        in_specs=[
            pl.BlockSpec((_BLK, C), lambda i: (i + blk0, 0)),
            pl.BlockSpec((4, _BLK, C), lambda i: (0, i, 0)),
            pl.BlockSpec((O, C), lambda i: (0, 0)),
            pl.BlockSpec((4, O, C), lambda i: (0, 0, 0)),
            pl.BlockSpec((O, 1), lambda i: (0, 0)),
        ],
        out_specs=pl.BlockSpec((O, _BLK), lambda i: (0, i)),
        out_shape=jax.ShapeDtypeStruct((O, Es), jnp.float32),
    )(xT, f4, w0, w14, b2)


def kernel(x, gemm_edges, W, b):
    x2d = x[0, :, :, 0]
    E = x2d.shape[1]
    xT = jnp.transpose(x2d)
    ge = gemm_edges[0].astype(jnp.int32)
    i1, i2, i3, i4 = ge[:, 0], ge[:, 1], ge[:, 2], ge[:, 3]
    w0 = W[:, :, 0, 0].astype(jnp.bfloat16)
    w14 = jnp.transpose(
        W[:, :, 0, 1:], (2, 0, 1)).astype(jnp.bfloat16)
    b2 = b[:, None]
    Es = E // _S
    f4s = [_sc_gather(xT, i1, i2, i3, i4, s * Es, Es) for s in range(_S)]
    outs = [_tc_conv(xT, f4s[s], w0, w14, b2, s * Es, Es)
            for s in range(_S)]
    out = jnp.concatenate(outs, axis=1) if _S > 1 else outs[0]
    return out[None, :, :, None]


# speedup vs baseline: 1.3129x; 1.0019x over previous
import functools

import jax
import jax.numpy as jnp
from jax import lax
from jax.experimental import pallas as pl
from jax.experimental.pallas import tpu as pltpu
from jax.experimental.pallas import tpu_sc as plsc

_NC, _NS = 2, 16
_NW = _NC * _NS
_S = 5
_BLK = 640


def _sc_gather(xT, i1, i2, i3, i4, e0, Es):
    E, C = xT.shape
    CH = 128
    n_chunks = Es // CH
    n_lo = n_chunks // _NW
    extra = n_chunks - n_lo * _NW
    max_ch = n_lo + (1 if extra else 0)

    mesh = plsc.VectorSubcoreMesh(core_axis_name="c", subcore_axis_name="s")

    @functools.partial(
        pl.kernel,
        mesh=mesh,
        out_type=jax.ShapeDtypeStruct((4, Es, C), jnp.float32),
        scratch_types=[
            pltpu.VMEM((max_ch * CH,), jnp.int32),
            pltpu.VMEM((max_ch * CH,), jnp.int32),
            pltpu.VMEM((max_ch * CH,), jnp.int32),
            pltpu.VMEM((max_ch * CH,), jnp.int32),
            pltpu.VMEM((4, CH, C), jnp.float32),
            pltpu.SemaphoreType.DMA,
        ],
    )
    def k(xT_hbm, i1_hbm, i2_hbm, i3_hbm, i4_hbm, out_hbm,
          iv1, iv2, iv3, iv4, rows_v, sem):
        wid = lax.axis_index("s") * _NC + lax.axis_index("c")
        start_ch = n_lo * wid + jnp.minimum(wid, extra)
        n_ch = n_lo + jnp.where(wid < extra, 1, 0)
        base = pl.multiple_of(start_ch * CH, CH)
        idx_vs = (iv1, iv2, iv3, iv4)
        for j4, ik in enumerate((i1_hbm, i2_hbm, i3_hbm, i4_hbm)):
            pltpu.sync_copy(ik.at[pl.ds(e0 + base, n_lo * CH)],
                            idx_vs[j4].at[pl.ds(0, n_lo * CH)])
        if extra:
            @pl.when(wid < extra)
            def _():
                for j4, ik in enumerate((i1_hbm, i2_hbm, i3_hbm, i4_hbm)):
                    pltpu.sync_copy(
                        ik.at[pl.ds(e0 + base + n_lo * CH, CH)],
                        idx_vs[j4].at[pl.ds(n_lo * CH, CH)])

        def chunk(j, carry):
            off = pl.multiple_of(j * CH, CH)
            cps = [
                pltpu.async_copy(
                    xT_hbm.at[idx_vs[j4].at[pl.ds(off, CH)]], rows_v.at[j4],
                    sem,
                )
                for j4 in range(4)
            ]
            for cp in cps:
                cp.wait()
            for j4 in range(4):
                pltpu.sync_copy(
                    rows_v.at[j4], out_hbm.at[j4, pl.ds(base + off, CH)]
                )
            return carry

        lax.fori_loop(0, n_ch, chunk, 0)

    return k(xT, i1, i2, i3, i4)


def _tc_conv(xT, f4, w0, w14, b2, e0, Es):
    E, C = xT.shape
    O = w0.shape[0]
    blk0 = e0 // _BLK

    def body(x_ref, f_ref, w0_ref, w14_ref, b_ref, o_ref):
        bf = jnp.bfloat16
        xb = x_ref[...].astype(bf)
        f1 = f_ref[0]
        f2 = f_ref[1]
        f3 = f_ref[2]
        f4_ = f_ref[3]
        s13 = (f1 + f3).astype(bf)
        s24 = (f2 + f4_).astype(bf)
        d13 = jnp.abs(f1 - f3).astype(bf)
        d24 = jnp.abs(f2 - f4_).astype(bf)
        w14 = w14_ref[...]

        def mm(wk, feat):
            return lax.dot_general(
                wk, feat, (((1,), (1,)), ((), ())),
                preferred_element_type=jnp.float32,
            )

        acc = (mm(w0_ref[...], xb) + mm(w14[0], s13) + mm(w14[1], s24)
               + mm(w14[2], d13) + mm(w14[3], d24))
        o_ref[...] = acc + b_ref[...]

    return pl.pallas_call(
        body,
        grid=(Es // _BLK,),
        in_specs=[
            pl.BlockSpec((_BLK, C), lambda i: (i + blk0, 0)),
            pl.BlockSpec((4, _BLK, C), lambda i: (0, i, 0)),
            pl.BlockSpec((O, C), lambda i: (0, 0)),
            pl.BlockSpec((4, O, C), lambda i: (0, 0, 0)),
            pl.BlockSpec((O, 1), lambda i: (0, 0)),
        ],
        out_specs=pl.BlockSpec((O, _BLK), lambda i: (0, i)),
        out_shape=jax.ShapeDtypeStruct((O, Es), jnp.float32),
    )(xT, f4, w0, w14, b2)


def kernel(x, gemm_edges, W, b):
    x2d = x[0, :, :, 0]
    E = x2d.shape[1]
    xT = jnp.transpose(x2d)
    ge = gemm_edges[0].astype(jnp.int32)
    i1, i2, i3, i4 = ge[:, 0], ge[:, 1], ge[:, 2], ge[:, 3]
    w0 = W[:, :, 0, 0].astype(jnp.bfloat16)
    w14 = jnp.transpose(
        W[:, :, 0, 1:], (2, 0, 1)).astype(jnp.bfloat16)
    b2 = b[:, None]
    Es = E // _S
    f4s = [_sc_gather(xT, i1, i2, i3, i4, s * Es, Es) for s in range(_S)]
    outs = [_tc_conv(xT, f4s[s], w0, w14, b2, s * Es, Es)
            for s in range(_S)]
    out = jnp.concatenate(outs, axis=1) if _S > 1 else outs[0]
    return out[None, :, :, None]
